# Initial kernel scaffold; baseline (speedup 1.0000x reference)
import jax
import jax.numpy as jnp
from jax.experimental import pallas as pl


def kernel(x, edge_index, edge_attr, batch, params):
    raise NotImplementedError("write your pallas kernel here")


# speedup vs baseline: 2.6649x; 2.6649x over previous
import functools

import jax
import jax.numpy as jnp
from jax import lax
from jax.experimental import pallas as pl
from jax.experimental.pallas import tpu as pltpu
from jax.experimental.pallas import tpu_sc as plsc

_N = 10000
_E = 320000
_ZD = 128
_NG = 64
_NC = 2
_NS = 16
_EPW = _E // (_NC * _NS)
_C = 80
_NCHUNK = _EPW // _C
_RPT = 624
_ZR = 156
_TAIL = _N - _RPT * _NS
_F32 = jnp.float32


def _matmul_bias_kernel(x_ref, w_ref, b_ref, o_ref):
    o_ref[...] = (
        jnp.dot(x_ref[...], w_ref[...], preferred_element_type=_F32, precision=lax.Precision.HIGHEST)
        + b_ref[...]
    )


def _node_proj(x, w, b):
    return pl.pallas_call(
        _matmul_bias_kernel,
        out_shape=jax.ShapeDtypeStruct((_N, _ZD), _F32),
    )(x, w, b.reshape(1, _ZD))


def _edge_proj(edge_attr, w, b):
    BE = 4000
    return pl.pallas_call(
        _matmul_bias_kernel,
        grid=(_E // BE,),
        in_specs=[
            pl.BlockSpec((BE, 16), lambda i: (i, 0)),
            pl.BlockSpec((16, _ZD), lambda i: (0, 0)),
            pl.BlockSpec((1, _ZD), lambda i: (0, 0)),
        ],
        out_specs=pl.BlockSpec((BE, _ZD), lambda i: (i, 0)),
        out_shape=jax.ShapeDtypeStruct((_E, _ZD), _F32),
    )(edge_attr, w, b.reshape(1, _ZD))


def _segment_onehot(batch_row):
    gids = lax.broadcasted_iota(jnp.int32, (_NG, _N), 0)
    return (batch_row == gids).astype(_F32)


def _mlp_kernel(h_ref, agg_ref, seps_ref, w1_ref, b1_ref, w2_ref, b2_ref,
                o_ref):
    z = h_ref[...] * seps_ref[...] + agg_ref[0] + agg_ref[1]
    a = jnp.maximum(
        jnp.dot(z, w1_ref[...], preferred_element_type=_F32, precision=lax.Precision.HIGHEST) + b1_ref[...],
        0.0)
    o_ref[...] = jnp.dot(a, w2_ref[...], preferred_element_type=_F32, precision=lax.Precision.HIGHEST) + b2_ref[...]


def _gnorm_kernel(u_ref, batch_ref, gw_ref, gb_ref, gms_ref, o_ref):
    P = _segment_onehot(batch_ref[...])
    u = u_ref[...]
    S1 = jnp.dot(P, u, preferred_element_type=_F32, precision=lax.Precision.HIGHEST)
    cnt = jnp.sum(P, axis=1, keepdims=True)
    mean = S1 / cnt
    ms = gms_ref[...]
    mm = ms * mean
    dn = (((0,), (0,)), ((), ()))
    mean_b = lax.dot_general(P, mm, dn, preferred_element_type=_F32, precision=lax.Precision.HIGHEST)
    sub = u - mean_b
    V = jnp.dot(P, sub * sub, preferred_element_type=_F32, precision=lax.Precision.HIGHEST)
    var = V / cnt
    var_b = lax.dot_general(P, var, dn, preferred_element_type=_F32, precision=lax.Precision.HIGHEST)
    o_ref[...] = jnp.maximum(
        gw_ref[...] * sub * lax.rsqrt(var_b + 1e-5) + gb_ref[...], 0.0)


def _update(h, agg, batch_row, seps, lp):
    BR = 2000
    u = pl.pallas_call(
        _mlp_kernel,
        grid=(_N // BR,),
        in_specs=[
            pl.BlockSpec((BR, _ZD), lambda i: (i, 0)),
            pl.BlockSpec((2, BR, _ZD), lambda i: (0, i, 0)),
            pl.BlockSpec((1, _ZD), lambda i: (0, 0)),
            pl.BlockSpec((_ZD, _ZD), lambda i: (0, 0)),
            pl.BlockSpec((1, _ZD), lambda i: (0, 0)),
            pl.BlockSpec((_ZD, _ZD), lambda i: (0, 0)),
            pl.BlockSpec((1, _ZD), lambda i: (0, 0)),
        ],
        out_specs=pl.BlockSpec((BR, _ZD), lambda i: (i, 0)),
        out_shape=jax.ShapeDtypeStruct((_N, _ZD), _F32),
    )(h, agg, seps,
      lp["mlp1"]["w"], lp["mlp1"]["b"].reshape(1, _ZD),
      lp["mlp2"]["w"], lp["mlp2"]["b"].reshape(1, _ZD))
    return pl.pallas_call(
        _gnorm_kernel,
        out_shape=jax.ShapeDtypeStruct((_N, _ZD), _F32),
    )(u, batch_row,
      lp["gn_w"].reshape(1, _ZD), lp["gn_b"].reshape(1, _ZD),
      lp["gn_ms"].reshape(1, _ZD))


def _heads_kernel(h_ref, batch_ref, wi1, bi1, wi2, bi2, ws1, bs1, ws2, bs2,
                  wp1, bp1, wp2, bp2, wa1, ba1, wa2, ba2,
                  hg_ref, zi_ref, zs_ref, y_ref, env_ref):
    P = _segment_onehot(batch_ref[...])
    cnt = jnp.sum(P, axis=1, keepdims=True)
    hg = jnp.dot(P, h_ref[...], preferred_element_type=_F32, precision=lax.Precision.HIGHEST) / cnt
    hg_ref[...] = hg

    def mlp(v, wa, ba, wb, bb):
        t = jnp.maximum(
            jnp.dot(v, wa[...], preferred_element_type=_F32, precision=lax.Precision.HIGHEST) + ba[...], 0.0)
        return jnp.dot(t, wb[...], preferred_element_type=_F32, precision=lax.Precision.HIGHEST) + bb[...]

    zi = mlp(hg, wi1, bi1, wi2, bi2)
    zi_ref[...] = zi
    zs_ref[...] = mlp(hg, ws1, bs1, ws2, bs2)
    y_ref[...] = mlp(zi, wp1, bp1, wp2, bp2)
    env_ref[...] = mlp(zi, wa1, ba1, wa2, ba2)


def _heads(h, batch_row, params):
    p = params
    outs = pl.pallas_call(
        _heads_kernel,
        out_shape=(
            jax.ShapeDtypeStruct((_NG, _ZD), _F32),
            jax.ShapeDtypeStruct((_NG, 64), _F32),
            jax.ShapeDtypeStruct((_NG, 64), _F32),
            jax.ShapeDtypeStruct((_NG, 1), _F32),
            jax.ShapeDtypeStruct((_NG, 4), _F32),
        ),
    )(h, batch_row,
      p["f_inv1"]["w"], p["f_inv1"]["b"].reshape(1, -1),
      p["f_inv2"]["w"], p["f_inv2"]["b"].reshape(1, -1),
      p["f_spu1"]["w"], p["f_spu1"]["b"].reshape(1, -1),
      p["f_spu2"]["w"], p["f_spu2"]["b"].reshape(1, -1),
      p["pred1"]["w"], p["pred1"]["b"].reshape(1, -1),
      p["pred2"]["w"], p["pred2"]["b"].reshape(1, -1),
      p["adv1"]["w"], p["adv1"]["b"].reshape(1, -1),
      p["adv2"]["w"], p["adv2"]["b"].reshape(1, -1))
    return outs


def _sc_body(h_hbm, e_hbm, src_hbm, dst_hbm, out_hbm,
             sidx, didx, hrows, erows, zbuf, accum, sem):
    c = lax.axis_index("c")
    s = lax.axis_index("s")

    @pl.loop(0, _ZR)
    def _zero_stage(i):
        for j in range(_ZD // 16):
            zbuf[i, pl.ds(j * 16, 16)] = jnp.zeros((16,), _F32)

    rbase = s * _RPT

    @pl.loop(0, _RPT // _ZR)
    def _zero_acc(i):
        pltpu.sync_copy(zbuf, accum.at[pl.ds(rbase + i * _ZR, _ZR)])

    @pl.when(s == _NS - 1)
    def _zero_tail():
        pltpu.sync_copy(zbuf.at[pl.ds(0, _TAIL)],
                        accum.at[pl.ds(_NS * _RPT, _TAIL)])

    plsc.subcore_barrier()

    base = (c * _NS + s) * _EPW

    @pl.loop(0, _NCHUNK)
    def _edge_chunk(i):
        off = base + i * _C
        pltpu.sync_copy(src_hbm.at[pl.ds(off, _C)], sidx)
        pltpu.sync_copy(dst_hbm.at[pl.ds(off, _C)], didx)
        pltpu.async_copy(h_hbm.at[sidx], hrows, sem).wait()
        pltpu.sync_copy(e_hbm.at[pl.ds(off, _C)], erows)

        @pl.loop(0, _C)
        def _row(r):
            for j in range(_ZD // 16):
                sl = pl.ds(j * 16, 16)
                erows[r, sl] = jnp.maximum(erows[r, sl] + hrows[r, sl], 0.0)

        pltpu.sync_copy(erows, accum.at[didx], add=True)

    plsc.subcore_barrier()
    pltpu.sync_copy(accum.at[pl.ds(rbase, _RPT)],
                    out_hbm.at[c, pl.ds(rbase, _RPT)])

    @pl.when(s == _NS - 1)
    def _write_tail():
        pltpu.sync_copy(accum.at[pl.ds(_NS * _RPT, _TAIL)],
                        out_hbm.at[c, pl.ds(_NS * _RPT, _TAIL)])


def _sc_aggregate(h, e, src, dst):
    mesh = plsc.VectorSubcoreMesh(core_axis_name="c", subcore_axis_name="s")

    f = pl.kernel(
        _sc_body,
        out_type=jax.ShapeDtypeStruct((_NC, _N, _ZD), _F32),
        mesh=mesh,
        scratch_types=[
            pltpu.VMEM((_C,), jnp.int32),
            pltpu.VMEM((_C,), jnp.int32),
            pltpu.VMEM((_C, _ZD), _F32),
            pltpu.VMEM((_C, _ZD), _F32),
            pltpu.VMEM((_ZR, _ZD), _F32),
            pltpu.VMEM_SHARED((_N, _ZD), _F32),
            pltpu.SemaphoreType.DMA,
        ],
    )
    return f(h, e, src, dst)


def kernel(x, edge_index, edge_attr, batch, params):
    src = edge_index[0]
    dst = edge_index[1]
    batch_row = batch.reshape(1, _N)

    h = _node_proj(x, params["node_proj"]["w"], params["node_proj"]["b"])
    for lp in params["layers"]:
        e = _edge_proj(edge_attr, lp["lin_edge"]["w"], lp["lin_edge"]["b"])
        agg = _sc_aggregate(h, e, src, dst)
        seps = jnp.full((1, _ZD), 1.0 + lp["eps"], _F32)
        h = _update(h, agg, batch_row, seps, lp)

    hg, z_inv, z_spu, yhat2, envhat = _heads(h, batch_row, params)
    return (hg, z_inv, z_spu, yhat2[:, 0], envhat)


# speedup vs baseline: 5.0317x; 1.8882x over previous
import functools

import jax
import jax.numpy as jnp
from jax import lax
from jax.experimental import pallas as pl
from jax.experimental.pallas import tpu as pltpu
from jax.experimental.pallas import tpu_sc as plsc

_N = 10000
_E = 320000
_ZD = 128
_NG = 64
_NC = 2
_NS = 16
_EPW = _E // (_NC * _NS)
_NPASS = 5
_EPP = _EPW // _NPASS
_C = 40
_NCHUNK = _EPP // _C
_NPAIR = _NCHUNK // 2
_RPT = 624
_TAIL = _N - _RPT * _NS
_F32 = jnp.float32
_HIGH = lax.Precision.HIGHEST
_LIN = lax.Precision.DEFAULT


def _matmul_bias_kernel(x_ref, w_ref, b_ref, o_ref):
    o_ref[...] = (jnp.dot(x_ref[...], w_ref[...], preferred_element_type=_F32,
                          precision=_LIN) + b_ref[...])


def _node_proj(x, w, b):
    return pl.pallas_call(
        _matmul_bias_kernel,
        out_shape=jax.ShapeDtypeStruct((_N, _ZD), _F32),
    )(x, w, b.reshape(1, _ZD))


def _edge_proj(edge_attr, w, b):
    BE = 4000
    return pl.pallas_call(
        _matmul_bias_kernel,
        grid=(_E // BE,),
        in_specs=[
            pl.BlockSpec((BE, 16), lambda i: (i, 0)),
            pl.BlockSpec((16, _ZD), lambda i: (0, 0)),
            pl.BlockSpec((1, _ZD), lambda i: (0, 0)),
        ],
        out_specs=pl.BlockSpec((BE, _ZD), lambda i: (i, 0)),
        out_shape=jax.ShapeDtypeStruct((_E, _ZD), _F32),
    )(edge_attr, w, b.reshape(1, _ZD))


def _segment_onehot(batch_row):
    gids = lax.broadcasted_iota(jnp.int32, (_NG, _N), 0)
    return (batch_row == gids).astype(_F32)


def _mlp_kernel(h_ref, agg_ref, seps_ref, w1_ref, b1_ref, w2_ref, b2_ref,
                o_ref):
    z = h_ref[...] * seps_ref[...] + agg_ref[0] + agg_ref[1]
    a = jnp.maximum(
        jnp.dot(z, w1_ref[...], preferred_element_type=_F32,
                precision=_LIN) + b1_ref[...], 0.0)
    o_ref[...] = (jnp.dot(a, w2_ref[...], preferred_element_type=_F32,
                          precision=_LIN) + b2_ref[...])


def _gnorm_kernel(u_ref, batch_ref, gw_ref, gb_ref, gms_ref, o_ref):
    P = _segment_onehot(batch_ref[...])
    u = u_ref[...]
    S1 = jnp.dot(P, u, preferred_element_type=_F32, precision=_HIGH)
    cnt = jnp.sum(P, axis=1, keepdims=True)
    mean = S1 / cnt
    ms = gms_ref[...]
    mm = ms * mean
    dn = (((0,), (0,)), ((), ()))
    mean_b = lax.dot_general(P, mm, dn, preferred_element_type=_F32,
                             precision=_HIGH)
    sub = u - mean_b
    V = jnp.dot(P, sub * sub, preferred_element_type=_F32, precision=_HIGH)
    var = V / cnt
    var_b = lax.dot_general(P, var, dn, preferred_element_type=_F32,
                            precision=_HIGH)
    o_ref[...] = jnp.maximum(
        gw_ref[...] * sub / jnp.sqrt(var_b + 1e-5) + gb_ref[...], 0.0)


def _update(h, agg, batch_row, seps, lp):
    BR = 2000
    u = pl.pallas_call(
        _mlp_kernel,
        grid=(_N // BR,),
        in_specs=[
            pl.BlockSpec((BR, _ZD), lambda i: (i, 0)),
            pl.BlockSpec((2, BR, _ZD), lambda i: (0, i, 0)),
            pl.BlockSpec((1, _ZD), lambda i: (0, 0)),
            pl.BlockSpec((_ZD, _ZD), lambda i: (0, 0)),
            pl.BlockSpec((1, _ZD), lambda i: (0, 0)),
            pl.BlockSpec((_ZD, _ZD), lambda i: (0, 0)),
            pl.BlockSpec((1, _ZD), lambda i: (0, 0)),
        ],
        out_specs=pl.BlockSpec((BR, _ZD), lambda i: (i, 0)),
        out_shape=jax.ShapeDtypeStruct((_N, _ZD), _F32),
    )(h, agg, seps,
      lp["mlp1"]["w"], lp["mlp1"]["b"].reshape(1, _ZD),
      lp["mlp2"]["w"], lp["mlp2"]["b"].reshape(1, _ZD))
    return pl.pallas_call(
        _gnorm_kernel,
        out_shape=jax.ShapeDtypeStruct((_N, _ZD), _F32),
    )(u, batch_row,
      lp["gn_w"].reshape(1, _ZD), lp["gn_b"].reshape(1, _ZD),
      lp["gn_ms"].reshape(1, _ZD))


def _heads_kernel(h_ref, batch_ref,
                  wi1, bi1, wi2, bi2, ws1, bs1, ws2, bs2,
                  wp1, bp1, wp2, bp2, wa1, ba1, wa2, ba2,
                  hg_ref, zi_ref, zs_ref, y_ref, env_ref):
    h = h_ref[...]
    P = _segment_onehot(batch_ref[...])
    cnt = jnp.sum(P, axis=1, keepdims=True)
    hg = jnp.dot(P, h, preferred_element_type=_F32, precision=_HIGH) / cnt
    hg_ref[...] = hg

    def mlp(v, wa, ba, wb, bb):
        t = jnp.maximum(
            jnp.dot(v, wa[...], preferred_element_type=_F32,
                    precision=_LIN) + ba[...], 0.0)
        return jnp.dot(t, wb[...], preferred_element_type=_F32,
                       precision=_LIN) + bb[...]

    zi = mlp(hg, wi1, bi1, wi2, bi2)
    zi_ref[...] = zi
    zs_ref[...] = mlp(hg, ws1, bs1, ws2, bs2)
    y_ref[...] = mlp(zi, wp1, bp1, wp2, bp2)
    env_ref[...] = mlp(zi, wa1, ba1, wa2, ba2)


def _heads(h, batch_row, params):
    p = params
    return pl.pallas_call(
        _heads_kernel,
        out_shape=(
            jax.ShapeDtypeStruct((_NG, _ZD), _F32),
            jax.ShapeDtypeStruct((_NG, 64), _F32),
            jax.ShapeDtypeStruct((_NG, 64), _F32),
            jax.ShapeDtypeStruct((_NG, 1), _F32),
            jax.ShapeDtypeStruct((_NG, 4), _F32),
        ),
    )(h, batch_row,
      p["f_inv1"]["w"], p["f_inv1"]["b"].reshape(1, -1),
      p["f_inv2"]["w"], p["f_inv2"]["b"].reshape(1, -1),
      p["f_spu1"]["w"], p["f_spu1"]["b"].reshape(1, -1),
      p["f_spu2"]["w"], p["f_spu2"]["b"].reshape(1, -1),
      p["pred1"]["w"], p["pred1"]["b"].reshape(1, -1),
      p["pred2"]["w"], p["pred2"]["b"].reshape(1, -1),
      p["adv1"]["w"], p["adv1"]["b"].reshape(1, -1),
      p["adv2"]["w"], p["adv2"]["b"].reshape(1, -1))


def _sc_body(h_hbm, e_hbm, src_hbm, dst_hbm, out_hbm,
             sidx_all, didx_all, hrA, erA, hrB, erB, msgA, msgB,
             didxA, didxB,
             gsemA, gsemB, esemA, esemB, sctA, sctB):
    c = lax.axis_index("c")
    s = lax.axis_index("s")
    wbase = (c * _NS + s) * _EPW
    accum = _sc_body._accum

    @pl.loop(0, _C)
    def _zero_stage(i):
        for j in range(_ZD // 16):
            msgA[i, pl.ds(j * 16, 16)] = jnp.zeros((16,), _F32)

    rbase = s * _RPT

    @pl.loop(0, 26)
    def _zero_acc(i):
        pltpu.sync_copy(msgA.at[pl.ds(0, 24)],
                        accum.at[pl.ds(rbase + i * 24, 24)])

    @pl.when(s == _NS - 1)
    def _zero_tail():
        pltpu.sync_copy(msgA.at[pl.ds(0, _TAIL)],
                        accum.at[pl.ds(_NS * _RPT, _TAIL)])

    plsc.subcore_barrier()

    def run_pass(base):
        pltpu.sync_copy(src_hbm.at[pl.ds(base, _EPP)], sidx_all)
        pltpu.sync_copy(dst_hbm.at[pl.ds(base, _EPP)], didx_all)

        def issue_loads(i, hr, er, gsem, esem):
            pltpu.async_copy(h_hbm.at[sidx_all.at[pl.ds(i * _C, _C)]],
                             hr, gsem)
            pltpu.async_copy(e_hbm.at[pl.ds(base + i * _C, _C)], er, esem)

        def wait_loads(i, hr, er, gsem, esem):
            pltpu.make_async_copy(
                h_hbm.at[sidx_all.at[pl.ds(i * _C, _C)]], hr, gsem).wait()
            pltpu.make_async_copy(
                e_hbm.at[pl.ds(base + i * _C, _C)], er, esem).wait()

        def compute(i, hr, er, msg, didx):
            for g in range(_C // 16):
                didx[pl.ds(g * 16, 16)] = \
                    didx_all[pl.ds(i * _C + g * 16, 16)]
            didx[pl.ds(_C - 16, 16)] = didx_all[pl.ds(i * _C + _C - 16, 16)]

            @pl.loop(0, _C)
            def _row(r):
                for j in range(_ZD // 16):
                    sl = pl.ds(j * 16, 16)
                    msg[r, sl] = jnp.maximum(hr[r, sl] + er[r, sl], 0.0)

        def issue_sct(msg, didx, sct):
            pltpu.make_async_copy(msg, accum.at[didx], sct).start(add=True)

        def wait_sct(msg, didx, sct):
            pltpu.make_async_copy(msg, accum.at[didx], sct).wait()

        issue_loads(0, hrA, erA, gsemA, esemA)
        issue_loads(1, hrB, erB, gsemB, esemB)

        @pl.loop(0, _NPAIR)
        def _pair(p):
            i = p * 2
            wait_loads(i, hrA, erA, gsemA, esemA)

            @pl.when(p > 0)
            def _():
                wait_sct(msgA, didxA, sctA)

            compute(i, hrA, erA, msgA, didxA)
            issue_sct(msgA, didxA, sctA)

            @pl.when(i + 2 < _NCHUNK)
            def _():
                issue_loads(i + 2, hrA, erA, gsemA, esemA)

            wait_loads(i + 1, hrB, erB, gsemB, esemB)

            @pl.when(p > 0)
            def _():
                wait_sct(msgB, didxB, sctB)

            compute(i + 1, hrB, erB, msgB, didxB)
            issue_sct(msgB, didxB, sctB)

            @pl.when(i + 3 < _NCHUNK)
            def _():
                issue_loads(i + 3, hrB, erB, gsemB, esemB)

        wait_sct(msgA, didxA, sctA)
        wait_sct(msgB, didxB, sctB)

    for p in range(_NPASS):
        run_pass(wbase + p * _EPP)

    plsc.subcore_barrier()
    pltpu.sync_copy(accum.at[pl.ds(rbase, _RPT)],
                    out_hbm.at[c, pl.ds(rbase, _RPT)])

    @pl.when(s == _NS - 1)
    def _write_tail():
        pltpu.sync_copy(accum.at[pl.ds(_NS * _RPT, _TAIL)],
                        out_hbm.at[c, pl.ds(_NS * _RPT, _TAIL)])


def _sc_kernel_entry(h_hbm, e_hbm, src_hbm, dst_hbm, out_hbm,
                     sidx_all, didx_all, hrA, erA, hrB, erB, msgA, msgB,
                     didxA, didxB, accum,
                     gsemA, gsemB, esemA, esemB, sctA, sctB):
    _sc_body._accum = accum
    _sc_body(h_hbm, e_hbm, src_hbm, dst_hbm, out_hbm,
             sidx_all, didx_all, hrA, erA, hrB, erB, msgA, msgB,
             didxA, didxB, gsemA, gsemB, esemA, esemB, sctA, sctB)


def _sc_aggregate(h, e, src, dst):
    mesh = plsc.VectorSubcoreMesh(core_axis_name="c", subcore_axis_name="s")

    f = pl.kernel(
        _sc_kernel_entry,
        out_type=jax.ShapeDtypeStruct((_NC, _N, _ZD), _F32),
        mesh=mesh,
        scratch_types=[
            pltpu.VMEM((_EPP,), jnp.int32),
            pltpu.VMEM((_EPP,), jnp.int32),
            pltpu.VMEM((_C, _ZD), _F32),
            pltpu.VMEM((_C, _ZD), _F32),
            pltpu.VMEM((_C, _ZD), _F32),
            pltpu.VMEM((_C, _ZD), _F32),
            pltpu.VMEM((_C, _ZD), _F32),
            pltpu.VMEM((_C, _ZD), _F32),
            pltpu.VMEM((_C,), jnp.int32),
            pltpu.VMEM((_C,), jnp.int32),
            pltpu.VMEM_SHARED((_N, _ZD), _F32),
            pltpu.SemaphoreType.DMA,
            pltpu.SemaphoreType.DMA,
            pltpu.SemaphoreType.DMA,
            pltpu.SemaphoreType.DMA,
            pltpu.SemaphoreType.DMA,
            pltpu.SemaphoreType.DMA,
        ],
    )
    return f(h, e, src, dst)


def kernel(x, edge_index, edge_attr, batch, params):
    src = edge_index[0]
    dst = edge_index[1]
    batch_row = batch.reshape(1, _N)

    h = _node_proj(x, params["node_proj"]["w"], params["node_proj"]["b"])
    for lp in params["layers"]:
        e = _edge_proj(edge_attr, lp["lin_edge"]["w"], lp["lin_edge"]["b"])
        agg = _sc_aggregate(h, e, src, dst)
        seps = jnp.full((1, _ZD), 1.0 + lp["eps"], _F32)
        h = _update(h, agg, batch_row, seps, lp)

    hg, z_inv, z_spu, yhat2, envhat = _heads(h, batch_row, params)
    return (hg, z_inv, z_spu, yhat2[:, 0], envhat)
